# Initial kernel scaffold; baseline (speedup 1.0000x reference)
import jax
import jax.numpy as jnp
from jax.experimental import pallas as pl


def kernel(x, edge_index, edge_attr, batch, en1_W, en1_b, en2_W, en2_b, en3_W, en3_b, conv1_root, conv1_bias, conv2_root, conv2_bias, conv3_root, conv3_bias, bn1_g, bn1_b, bn2_g, bn2_b, bn3_g, bn3_b, fc1_W, fc1_b, fc2_W, fc2_b):
    raise NotImplementedError("write your pallas kernel here")


# speedup vs baseline: 2.6972x; 2.6972x over previous
import functools

import jax
import jax.numpy as jnp
from jax import lax
from jax.experimental import pallas as pl
from jax.experimental.pallas import tpu as pltpu
from jax.experimental.pallas import tpu_sc as plsc

NW = 32
NS = 16
CHUNK = 125

_SC_PARAMS = pltpu.CompilerParams(use_tc_tiling_on_sc=False)


def _sc_gather(table, idx3, cin):
    nchunk = idx3.shape[1]
    ew = nchunk * CHUNK
    e_total = NW * ew
    mesh = plsc.VectorSubcoreMesh(core_axis_name="c", subcore_axis_name="s")

    group = 8
    grows = group * CHUNK
    ngroup = nchunk // group

    @functools.partial(
        pl.kernel,
        out_type=jax.ShapeDtypeStruct((e_total, cin), jnp.float32),
        mesh=mesh,
        scratch_types=[
            pltpu.VMEM((nchunk, CHUNK), jnp.int32),
            pltpu.VMEM((grows, cin), jnp.float32),
            pltpu.SemaphoreType.DMA,
        ],
        compiler_params=_SC_PARAMS,
    )
    def gk(table_hbm, idx_hbm, out_hbm, idx_v, rows_v, sem):
        c = lax.axis_index("c")
        s = lax.axis_index("s")
        wid = s * 2 + c
        pltpu.sync_copy(idx_hbm.at[wid], idx_v)

        def body(g, carry):
            def inner(j, carry2):
                pltpu.async_copy(table_hbm.at[idx_v.at[g * group + j]],
                                 rows_v.at[pl.ds(j * CHUNK, CHUNK)], sem).wait()
                return carry2

            lax.fori_loop(0, group, inner, 0)
            pltpu.sync_copy(rows_v,
                            out_hbm.at[pl.ds(wid * ew + g * grows, grows)])
            return carry

        lax.fori_loop(0, ngroup, body, 0)

    return gk(table, idx3)


def _sc_scatter(msg, dst3, n_nodes, cout):
    nchunk = dst3.shape[1]
    ew = nchunk * CHUNK
    rpt = n_nodes // NS
    mesh = plsc.VectorSubcoreMesh(core_axis_name="c", subcore_axis_name="s")

    @functools.partial(
        pl.kernel,
        out_type=jax.ShapeDtypeStruct((2, NS, rpt, cout), jnp.float32),
        mesh=mesh,
        scratch_types=[
            pltpu.VMEM((nchunk, CHUNK), jnp.int32),
            pltpu.VMEM((ew, cout), jnp.float32),
            pltpu.VMEM_SHARED((n_nodes, cout), jnp.float32),
        ],
        compiler_params=_SC_PARAMS,
    )
    def sk(msg_hbm, dst_hbm, out_hbm, dst_v, msg_v, acc):
        c = lax.axis_index("c")
        s = lax.axis_index("s")
        wid = s * 2 + c

        def zbody(i, carry):
            msg_v[i, :] = jnp.zeros((16,), jnp.float32)
            return carry

        lax.fori_loop(0, rpt, zbody, 0)
        pltpu.sync_copy(msg_v.at[pl.ds(0, rpt)], acc.at[pl.ds(s * rpt, rpt)])
        plsc.subcore_barrier()

        pltpu.sync_copy(dst_hbm.at[wid], dst_v)
        pltpu.sync_copy(msg_hbm.at[pl.ds(wid * ew, ew)], msg_v)

        def body(j, carry):
            pltpu.sync_copy(msg_v.at[pl.ds(j * CHUNK, CHUNK)],
                            acc.at[dst_v.at[j]], add=True)
            return carry

        lax.fori_loop(0, nchunk, body, 0)
        plsc.subcore_barrier()

        pltpu.sync_copy(acc.at[pl.ds(s * rpt, rpt)], msg_v.at[pl.ds(0, rpt)])
        pltpu.sync_copy(msg_v.at[pl.ds(0, rpt)], out_hbm.at[c, s])

    return sk(msg, dst3).reshape(2, n_nodes, cout)


def _tc_msg(ea, xj, wp, bp, cin, cout, te):
    e_total = ea.shape[0]
    fe = ea.shape[1]
    k = cin * cout

    def mk(ea_ref, xj_ref, w_ref, b_ref, out_ref):
        q = jnp.dot(ea_ref[...], w_ref[...], preferred_element_type=jnp.float32)
        q = jnp.maximum(q + b_ref[...], 0.0)
        xr = jnp.tile(xj_ref[...], (1, cout))
        sel = (lax.broadcasted_iota(jnp.int32, (k, cout), 0) // cin ==
               lax.broadcasted_iota(jnp.int32, (k, cout), 1)).astype(jnp.float32)
        out_ref[...] = jnp.dot(xr * q, sel, preferred_element_type=jnp.float32)

    return pl.pallas_call(
        mk,
        grid=(e_total // te,),
        in_specs=[
            pl.BlockSpec((te, fe), lambda i: (i, 0)),
            pl.BlockSpec((te, cin), lambda i: (i, 0)),
            pl.BlockSpec((fe, k), lambda i: (0, 0)),
            pl.BlockSpec((1, k), lambda i: (0, 0)),
        ],
        out_specs=pl.BlockSpec((te, cout), lambda i: (i, 0)),
        out_shape=jax.ShapeDtypeStruct((e_total, cout), jnp.float32),
    )(ea, xj, wp, bp)


def _tc_update(p, h, root, bias2, g2, b2):
    n = h.shape[0]
    cout = root.shape[1]

    def uk(p_ref, h_ref, r_ref, bias_ref, g_ref, b_ref, out_ref):
        z = (p_ref[0] + p_ref[1] + bias_ref[...] +
             jnp.dot(h_ref[...], r_ref[...], preferred_element_type=jnp.float32))
        m = jnp.mean(z, axis=0, keepdims=True)
        v = jnp.mean((z - m) ** 2, axis=0, keepdims=True)
        zn = (z - m) * lax.rsqrt(v + 1e-5) * g_ref[...] + b_ref[...]
        out_ref[...] = jnp.maximum(zn, 0.0)

    return pl.pallas_call(
        uk, out_shape=jax.ShapeDtypeStruct((n, cout), jnp.float32),
    )(p, h, root, bias2, g2, b2)


def _tc_update_head(p, h, root, bias2, g2, b2, batch2, n_graphs,
                    fc1_w, fc1_b2, fc2_w, fc2_b2):
    n = h.shape[0]

    def uk(p_ref, h_ref, r_ref, bias_ref, g_ref, b_ref, batch_ref,
           w1_ref, b1_ref, w2_ref, b2_ref, out_ref):
        z = (p_ref[0] + p_ref[1] + bias_ref[...] +
             jnp.dot(h_ref[...], r_ref[...], preferred_element_type=jnp.float32))
        m = jnp.mean(z, axis=0, keepdims=True)
        v = jnp.mean((z - m) ** 2, axis=0, keepdims=True)
        hh = jnp.maximum((z - m) * lax.rsqrt(v + 1e-5) * g_ref[...] + b_ref[...], 0.0)
        onehot = (batch_ref[...] ==
                  lax.broadcasted_iota(jnp.int32, (n_graphs, n), 0)
                  ).astype(jnp.float32)
        ssum = jnp.dot(onehot, hh, preferred_element_type=jnp.float32)
        cnt = jnp.sum(onehot, axis=1, keepdims=True)
        pooled = ssum / jnp.maximum(cnt, 1.0)
        o = jnp.maximum(
            jnp.dot(pooled, w1_ref[...], preferred_element_type=jnp.float32)
            + b1_ref[...], 0.0)
        out_ref[...] = (jnp.dot(o, w2_ref[...], preferred_element_type=jnp.float32)
                        + b2_ref[...])

    return pl.pallas_call(
        uk, out_shape=jax.ShapeDtypeStruct((n_graphs, 1), jnp.float32),
    )(p, h, root, bias2, g2, b2, batch2, fc1_w, fc1_b2, fc2_w, fc2_b2)


def _perm(w, b, cin, cout):
    fe = w.shape[0]
    wp = w.reshape(fe, cin, cout).transpose(0, 2, 1).reshape(fe, cin * cout)
    bp = b.reshape(cin, cout).T.reshape(1, cin * cout)
    return wp, bp


def kernel(x, edge_index, edge_attr, batch,
           en1_W, en1_b, en2_W, en2_b, en3_W, en3_b,
           conv1_root, conv1_bias, conv2_root, conv2_bias,
           conv3_root, conv3_bias,
           bn1_g, bn1_b, bn2_g, bn2_b, bn3_g, bn3_b,
           fc1_W, fc1_b, fc2_W, fc2_b):
    n, f_in = x.shape
    e_total = edge_index.shape[1]
    h_dim = conv1_root.shape[1]
    n_graphs = 64
    ew = e_total // NW
    nchunk = ew // CHUNK

    src3 = edge_index[0].reshape(NW, nchunk, CHUNK)
    dst3 = edge_index[1].reshape(NW, nchunk, CHUNK)
    batch2 = batch.reshape(1, n)

    wp1, bp1 = _perm(en1_W, en1_b, f_in, h_dim)
    wp2, bp2 = _perm(en2_W, en2_b, h_dim, h_dim)
    wp3, bp3 = _perm(en3_W, en3_b, h_dim, h_dim)

    def l2(v):
        return v.reshape(1, -1)

    h = x
    cin = f_in
    for (wp, bp, root, bias, g, b, te) in (
            (wp1, bp1, conv1_root, conv1_bias, bn1_g, bn1_b, 640),
            (wp2, bp2, conv2_root, conv2_bias, bn2_g, bn2_b, 1280),
            (wp3, bp3, conv3_root, conv3_bias, bn3_g, bn3_b, 1280),
    ):
        xj = _sc_gather(h, src3, cin)
        msg = _tc_msg(edge_attr, xj, wp, bp, cin, h_dim, te)
        p = _sc_scatter(msg, dst3, n, h_dim)
        if root is conv3_root:
            out = _tc_update_head(p, h, root, l2(bias), l2(g), l2(b),
                                  batch2, n_graphs,
                                  fc1_W, l2(fc1_b), fc2_W, l2(fc2_b))
        else:
            h = _tc_update(p, h, root, l2(bias), l2(g), l2(b))
            cin = h_dim
    return out
